# baseline (speedup 1.0000x reference)
import functools

import jax
import jax.numpy as jnp
from jax import lax
from jax.experimental import pallas as pl
from jax.experimental.pallas import tpu as pltpu

N = 320000
F = 128
S = 10000

BLK = 2000


def _mm_epilogue_kernel(x_ref, g_ref, w_ref, o_ref):
    acc = jnp.dot(x_ref[...], w_ref[...].T, preferred_element_type=jnp.float32)
    v = acc + g_ref[...]
    o_ref[...] = jnp.maximum(v, 0.01 * v)


def _mm_epilogue_res_kernel(x_ref, g_ref, w_ref, r_ref, o_ref):
    acc = jnp.dot(x_ref[...], w_ref[...].T, preferred_element_type=jnp.float32)
    v = acc + g_ref[...]
    v = jnp.maximum(v, 0.01 * v)
    v = r_ref[...] + v
    o_ref[...] = jnp.maximum(v, 0.01 * v)


def _fused_mm(x, g, Wx, residual=None):
    grid = (N // BLK,)
    row_spec = pl.BlockSpec((BLK, F), lambda i: (i, 0))
    w_spec = pl.BlockSpec((F, F), lambda i: (0, 0))
    if residual is None:
        return pl.pallas_call(
            _mm_epilogue_kernel,
            grid=grid,
            in_specs=[row_spec, row_spec, w_spec],
            out_specs=row_spec,
            out_shape=jax.ShapeDtypeStruct((N, F), jnp.float32),
        )(x, g, Wx)
    return pl.pallas_call(
        _mm_epilogue_res_kernel,
        grid=grid,
        in_specs=[row_spec, row_spec, w_spec, row_spec],
        out_specs=row_spec,
        out_shape=jax.ShapeDtypeStruct((N, F), jnp.float32),
    )(x, g, Wx, residual)


def _segment_sums(v, rows, cols):
    rs = jax.ops.segment_sum(v, rows, num_segments=S)
    cs = jax.ops.segment_sum(v, cols, num_segments=S)
    return rs, cs


def kernel(input, index, W1, b1, W2, b2):
    x = input
    rows = index[:, 0]
    cols = index[:, 1]
    ones = jnp.ones((N, 1), jnp.float32)
    cnt_r = jnp.clip(jax.ops.segment_sum(ones, rows, num_segments=S), 1.0)
    cnt_c = jnp.clip(jax.ops.segment_sum(ones, cols, num_segments=S), 1.0)

    def exchange(v, W, b):
        Wx, Wr, Wc, Wg = W[:, :F], W[:, F:2 * F], W[:, 2 * F:3 * F], W[:, 3 * F:]
        rsum, csum = _segment_sums(v, rows, cols)
        glob = jnp.sum(rsum, axis=0) / N
        Ta = (rsum / cnt_r) @ Wr.T + (glob @ Wg.T + b)[None, :]
        Tb = (csum / cnt_c) @ Wc.T
        g = Ta[rows] + Tb[cols]
        return g, Wx

    g1, Wx1 = exchange(x, W1, b1)
    step1 = _fused_mm(x, g1, Wx1)
    g2, Wx2 = exchange(step1, W2, b2)
    return _fused_mm(step1, g2, Wx2, residual=x)


# speedup vs baseline: 1.1484x; 1.1484x over previous
import functools

import jax
import jax.numpy as jnp
from jax import lax
from jax.experimental import pallas as pl
from jax.experimental.pallas import tpu as pltpu
from jax.experimental.pallas import tpu_sc as plsc

N = 320000
F = 128
S = 10000

BLK = 2000
C = 80
NS = 16
PER_TILE = N // NS
N_CHUNK = PER_TILE // C
SROWS = 640
SP = NS * SROWS


def _mm_epilogue_kernel(x_ref, g_ref, w_ref, o_ref):
    acc = jnp.dot(x_ref[...], w_ref[...].T, preferred_element_type=jnp.float32)
    v = acc + g_ref[...]
    o_ref[...] = jnp.maximum(v, 0.01 * v)


def _mm_epilogue_res_kernel(x_ref, g_ref, w_ref, r_ref, o_ref):
    acc = jnp.dot(x_ref[...], w_ref[...].T, preferred_element_type=jnp.float32)
    v = acc + g_ref[...]
    v = jnp.maximum(v, 0.01 * v)
    v = r_ref[...] + v
    o_ref[...] = jnp.maximum(v, 0.01 * v)


def _fused_mm(x, g, Wx, residual=None):
    grid = (N // BLK,)
    row_spec = pl.BlockSpec((BLK, F), lambda i: (i, 0))
    w_spec = pl.BlockSpec((F, F), lambda i: (0, 0))
    if residual is None:
        return pl.pallas_call(
            _mm_epilogue_kernel,
            grid=grid,
            in_specs=[row_spec, row_spec, w_spec],
            out_specs=row_spec,
            out_shape=jax.ShapeDtypeStruct((N, F), jnp.float32),
        )(x, g, Wx)
    return pl.pallas_call(
        _mm_epilogue_res_kernel,
        grid=grid,
        in_specs=[row_spec, row_spec, w_spec, row_spec],
        out_specs=row_spec,
        out_shape=jax.ShapeDtypeStruct((N, F), jnp.float32),
    )(x, g, Wx, residual)


def _make_scatter(with_counts: bool):
    mesh = plsc.VectorSubcoreMesh(core_axis_name="c", subcore_axis_name="s")
    out_type = [
        jax.ShapeDtypeStruct((SP, F), jnp.float32),
        jax.ShapeDtypeStruct((SP, F), jnp.float32),
    ]
    scratch = [
        pltpu.VMEM((C, F), jnp.float32),
        pltpu.VMEM((C,), jnp.int32),
        pltpu.VMEM_SHARED((SP, F), jnp.float32),
    ]
    if with_counts:
        out_type.append(jax.ShapeDtypeStruct((SP, 16), jnp.float32))
        out_type.append(jax.ShapeDtypeStruct((SP, 16), jnp.float32))
        scratch.append(pltpu.VMEM((C, 16), jnp.float32))
        scratch.append(pltpu.VMEM_SHARED((SP, 16), jnp.float32))

    def body(v_hbm, rows_hbm, cols_hbm, zf_hbm, zc_hbm, ones_hbm,
             rsum_hbm, csum_hbm, *rest):
        if with_counts:
            cntr_hbm, cntc_hbm, xbuf, ibuf, table, obuf, ctab = rest
        else:
            xbuf, ibuf, table = rest
        c = lax.axis_index("c")
        s = lax.axis_index("s")
        rs = pl.ds(s * SROWS, SROWS)

        pltpu.sync_copy(zf_hbm, table.at[rs, :])
        if with_counts:
            pltpu.sync_copy(zc_hbm, ctab.at[rs, :])
            pltpu.sync_copy(ones_hbm, obuf)
        plsc.subcore_barrier()

        tbase = s * PER_TILE

        def main_loop(ids_hbm):
            def step(it, carry):
                base = tbase + it * C
                pltpu.sync_copy(v_hbm.at[pl.ds(base, C), :], xbuf)
                pltpu.sync_copy(ids_hbm.at[pl.ds(base, C)], ibuf)
                pltpu.sync_copy(xbuf, table.at[ibuf], add=True)
                if with_counts:
                    pltpu.sync_copy(obuf, ctab.at[ibuf], add=True)
                return carry

            lax.fori_loop(0, N_CHUNK, step, 0)

        @pl.when(c == 0)
        def _():
            main_loop(rows_hbm)

        @pl.when(c == 1)
        def _():
            main_loop(cols_hbm)

        plsc.subcore_barrier()

        @pl.when(c == 0)
        def _():
            pltpu.sync_copy(table.at[rs, :], rsum_hbm.at[rs, :])
            if with_counts:
                pltpu.sync_copy(ctab.at[rs, :], cntr_hbm.at[rs, :])

        @pl.when(c == 1)
        def _():
            pltpu.sync_copy(table.at[rs, :], csum_hbm.at[rs, :])
            if with_counts:
                pltpu.sync_copy(ctab.at[rs, :], cntc_hbm.at[rs, :])

    return pl.kernel(body, out_type=out_type, mesh=mesh, scratch_types=scratch)


_ZF = functools.partial(jnp.zeros, (SROWS, F), jnp.float32)
_ZC = functools.partial(jnp.zeros, (SROWS, 16), jnp.float32)
_ONES = functools.partial(jnp.ones, (C, 16), jnp.float32)


def _segment_sums(v, rows, cols, with_counts):
    fn = _make_scatter(with_counts)
    return fn(v, rows, cols, _ZF(), _ZC(), _ONES())


def kernel(input, index, W1, b1, W2, b2):
    x = input
    rows = index[:, 0]
    cols = index[:, 1]

    def tables(rsum, csum, cnt_r, cnt_c, W, b):
        Wr, Wc, Wg = W[:, F:2 * F], W[:, 2 * F:3 * F], W[:, 3 * F:]
        glob = jnp.sum(rsum, axis=0) / N
        Ta = (rsum / cnt_r) @ Wr.T + (glob @ Wg.T + b)[None, :]
        Tb = (csum / cnt_c) @ Wc.T
        return Ta, Tb

    rsum1, csum1 = _segment_sums(x, rows, cols, False)
    rsum1, csum1 = rsum1[:S], csum1[:S]
    ones = jnp.ones((N,), jnp.float32)
    cnt_r = jnp.clip(jax.ops.segment_sum(ones, rows, num_segments=S), 1.0)[:, None]
    cnt_c = jnp.clip(jax.ops.segment_sum(ones, cols, num_segments=S), 1.0)[:, None]
    Ta1, Tb1 = tables(rsum1, csum1, cnt_r, cnt_c, W1, b1)
    g1 = Ta1[rows] + Tb1[cols]
    step1 = _fused_mm(x, g1, W1[:, :F])

    rsum2, csum2 = _segment_sums(step1, rows, cols, False)
    Ta2, Tb2 = tables(rsum2[:S], csum2[:S], cnt_r, cnt_c, W2, b2)
    g2 = Ta2[rows] + Tb2[cols]
    return _fused_mm(step1, g2, W2[:, :F], residual=x)


# speedup vs baseline: 1.9378x; 1.6873x over previous
import functools

import jax
import jax.numpy as jnp
from jax import lax
from jax.experimental import pallas as pl
from jax.experimental.pallas import tpu as pltpu
from jax.experimental.pallas import tpu_sc as plsc

N = 320000
F = 128
S = 10000

BLK = 2000
C = 80
NS = 16
PER_TILE = N // NS
N_CHUNK = PER_TILE // C
SROWS = 640
SP = NS * SROWS


def _mm_kernel(x_ref, w_ref, o_ref):
    o_ref[...] = jnp.dot(x_ref[...], w_ref[...].T,
                         preferred_element_type=jnp.float32)


def _mm(x, Wx):
    grid = (N // BLK,)
    row_spec = pl.BlockSpec((BLK, F), lambda i: (i, 0))
    w_spec = pl.BlockSpec((F, F), lambda i: (0, 0))
    return pl.pallas_call(
        _mm_kernel,
        grid=grid,
        in_specs=[row_spec, w_spec],
        out_specs=row_spec,
        out_shape=jax.ShapeDtypeStruct((N, F), jnp.float32),
    )(x, Wx)


def _make_scatter(with_counts: bool):
    mesh = plsc.VectorSubcoreMesh(core_axis_name="c", subcore_axis_name="s")
    out_type = [
        jax.ShapeDtypeStruct((SP, F), jnp.float32),
        jax.ShapeDtypeStruct((SP, F), jnp.float32),
    ]
    scratch = [
        pltpu.VMEM((C, F), jnp.float32),
        pltpu.VMEM((C,), jnp.int32),
        pltpu.VMEM_SHARED((SP, F), jnp.float32),
    ]
    if with_counts:
        out_type.append(jax.ShapeDtypeStruct((SP, 16), jnp.float32))
        out_type.append(jax.ShapeDtypeStruct((SP, 16), jnp.float32))
        scratch.append(pltpu.VMEM((C, 16), jnp.float32))
        scratch.append(pltpu.VMEM_SHARED((SP, 16), jnp.float32))

    def body(v_hbm, rows_hbm, cols_hbm, zf_hbm, zc_hbm, ones_hbm,
             rsum_hbm, csum_hbm, *rest):
        if with_counts:
            cntr_hbm, cntc_hbm, xbuf, ibuf, table, obuf, ctab = rest
        else:
            xbuf, ibuf, table = rest
        c = lax.axis_index("c")
        s = lax.axis_index("s")
        rs = pl.ds(s * SROWS, SROWS)

        pltpu.sync_copy(zf_hbm, table.at[rs, :])
        if with_counts:
            pltpu.sync_copy(zc_hbm, ctab.at[rs, :])
            pltpu.sync_copy(ones_hbm, obuf)
        plsc.subcore_barrier()

        tbase = s * PER_TILE

        def main_loop(ids_hbm):
            def step(it, carry):
                base = tbase + it * C
                pltpu.sync_copy(v_hbm.at[pl.ds(base, C), :], xbuf)
                pltpu.sync_copy(ids_hbm.at[pl.ds(base, C)], ibuf)
                pltpu.sync_copy(xbuf, table.at[ibuf], add=True)
                if with_counts:
                    pltpu.sync_copy(obuf, ctab.at[ibuf], add=True)
                return carry

            lax.fori_loop(0, N_CHUNK, step, 0)

        @pl.when(c == 0)
        def _():
            main_loop(rows_hbm)

        @pl.when(c == 1)
        def _():
            main_loop(cols_hbm)

        plsc.subcore_barrier()

        @pl.when(c == 0)
        def _():
            pltpu.sync_copy(table.at[rs, :], rsum_hbm.at[rs, :])
            if with_counts:
                pltpu.sync_copy(ctab.at[rs, :], cntr_hbm.at[rs, :])

        @pl.when(c == 1)
        def _():
            pltpu.sync_copy(table.at[rs, :], csum_hbm.at[rs, :])
            if with_counts:
                pltpu.sync_copy(ctab.at[rs, :], cntc_hbm.at[rs, :])

    return pl.kernel(body, out_type=out_type, mesh=mesh, scratch_types=scratch)


_ZF = functools.partial(jnp.zeros, (SROWS, F), jnp.float32)
_ZC = functools.partial(jnp.zeros, (SROWS, 16), jnp.float32)
_ONES = functools.partial(jnp.ones, (C, 16), jnp.float32)


def _segment_sums(v, rows, cols, with_counts):
    fn = _make_scatter(with_counts)
    return fn(v, rows, cols, _ZF(), _ZC(), _ONES())


GW = 32
G_PER = N // GW
G_CHUNK = G_PER // C


def _make_gather(with_residual: bool):
    mesh = plsc.VectorSubcoreMesh(core_axis_name="c", subcore_axis_name="s")
    out_type = jax.ShapeDtypeStruct((N, F), jnp.float32)
    scratch = [
        pltpu.VMEM((C,), jnp.int32),
        pltpu.VMEM((C,), jnp.int32),
        pltpu.VMEM((C, F), jnp.float32),
        pltpu.VMEM((C, F), jnp.float32),
        pltpu.VMEM((C, F), jnp.float32),
        pltpu.VMEM((C, F), jnp.float32),
        pltpu.SemaphoreType.DMA,
        pltpu.SemaphoreType.DMA,
    ]

    def body(xw_hbm, rows_hbm, cols_hbm, ta_hbm, tb_hbm, res_hbm, o_hbm,
             ribuf, cibuf, xbuf, abuf, bbuf, rbuf, sem_a, sem_b):
        c = lax.axis_index("c")
        s = lax.axis_index("s")
        w = c * NS + s
        tbase = w * G_PER

        def step(it, carry):
            base = tbase + it * C
            pltpu.sync_copy(rows_hbm.at[pl.ds(base, C)], ribuf)
            pltpu.sync_copy(cols_hbm.at[pl.ds(base, C)], cibuf)
            cp_a = pltpu.async_copy(ta_hbm.at[ribuf], abuf, sem_a)
            cp_b = pltpu.async_copy(tb_hbm.at[cibuf], bbuf, sem_b)
            pltpu.sync_copy(xw_hbm.at[pl.ds(base, C), :], xbuf)
            if with_residual:
                pltpu.sync_copy(res_hbm.at[pl.ds(base, C), :], rbuf)
            cp_a.wait()
            cp_b.wait()

            def row(i, carry2):
                for j in range(F // 16):
                    sl = pl.ds(j * 16, 16)
                    v = xbuf[i, sl] + abuf[i, sl] + bbuf[i, sl]
                    v = jnp.maximum(v, 0.01 * v)
                    if with_residual:
                        v = rbuf[i, sl] + v
                        v = jnp.maximum(v, 0.01 * v)
                    xbuf[i, sl] = v
                return carry2

            lax.fori_loop(0, C, row, 0)
            pltpu.sync_copy(xbuf, o_hbm.at[pl.ds(base, C), :])
            return carry

        lax.fori_loop(0, G_CHUNK, step, 0)

    return pl.kernel(body, out_type=out_type, mesh=mesh, scratch_types=scratch)


def _gather_epilogue(xw, rows, cols, Ta, Tb, residual):
    fn = _make_gather(residual is not None)
    res = xw if residual is None else residual
    return fn(xw, rows, cols, Ta, Tb, res)


def kernel(input, index, W1, b1, W2, b2):
    x = input
    rows = index[:, 0]
    cols = index[:, 1]

    def tables(rsum, csum, cnt_r, cnt_c, W, b):
        Wr, Wc, Wg = W[:, F:2 * F], W[:, 2 * F:3 * F], W[:, 3 * F:]
        glob = jnp.sum(rsum, axis=0) / N
        Ta = (rsum / cnt_r) @ Wr.T + (glob @ Wg.T + b)[None, :]
        Tb = (csum / cnt_c) @ Wc.T
        return Ta, Tb

    rsum1, csum1 = _segment_sums(x, rows, cols, False)
    rsum1, csum1 = rsum1[:S], csum1[:S]
    ones = jnp.ones((N,), jnp.float32)
    cnt_r = jnp.clip(jax.ops.segment_sum(ones, rows, num_segments=S), 1.0)[:, None]
    cnt_c = jnp.clip(jax.ops.segment_sum(ones, cols, num_segments=S), 1.0)[:, None]
    Ta1, Tb1 = tables(rsum1, csum1, cnt_r, cnt_c, W1, b1)
    xw1 = _mm(x, W1[:, :F])
    step1 = _gather_epilogue(xw1, rows, cols, Ta1, Tb1, None)

    rsum2, csum2 = _segment_sums(step1, rows, cols, False)
    Ta2, Tb2 = tables(rsum2[:S], csum2[:S], cnt_r, cnt_c, W2, b2)
    xw2 = _mm(step1, W2[:, :F])
    return _gather_epilogue(xw2, rows, cols, Ta2, Tb2, x)


# speedup vs baseline: 2.8414x; 1.4663x over previous
import functools

import jax
import jax.numpy as jnp
from jax import lax
from jax.experimental import pallas as pl
from jax.experimental.pallas import tpu as pltpu
from jax.experimental.pallas import tpu_sc as plsc

N = 320000
F = 128
S = 10000

BLK = 2000
C = 80
NS = 16
PER_TILE = N // NS
N_CHUNK = PER_TILE // C
SROWS = 640
SP = NS * SROWS


def _mm_kernel(x_ref, w_ref, o_ref):
    o_ref[...] = jnp.dot(x_ref[...], w_ref[...].T,
                         preferred_element_type=jnp.float32)


def _mm(x, Wx):
    grid = (N // BLK,)
    row_spec = pl.BlockSpec((BLK, F), lambda i: (i, 0))
    w_spec = pl.BlockSpec((F, F), lambda i: (0, 0))
    return pl.pallas_call(
        _mm_kernel,
        grid=grid,
        in_specs=[row_spec, w_spec],
        out_specs=row_spec,
        out_shape=jax.ShapeDtypeStruct((N, F), jnp.float32),
    )(x, Wx)


def _make_scatter_db():
    mesh = plsc.VectorSubcoreMesh(core_axis_name="c", subcore_axis_name="s")
    out_type = [
        jax.ShapeDtypeStruct((SP, F), jnp.float32),
        jax.ShapeDtypeStruct((SP, F), jnp.float32),
    ]
    scratch = (
        [pltpu.VMEM((C, F), jnp.float32)] * 2
        + [pltpu.VMEM((C,), jnp.int32)] * 2
        + [pltpu.VMEM_SHARED((SP, F), jnp.float32)]
        + [pltpu.SemaphoreType.DMA] * 6
    )

    def body(v_hbm, rows_hbm, cols_hbm, zf_hbm, rsum_hbm, csum_hbm,
             xb0, xb1, ib0, ib1, table, sx0, sx1, si0, si1, ss0, ss1):
        c = lax.axis_index("c")
        s = lax.axis_index("s")
        rs = pl.ds(s * SROWS, SROWS)
        xb, ib = (xb0, xb1), (ib0, ib1)
        sx, si, ss = (sx0, sx1), (si0, si1), (ss0, ss1)

        pltpu.sync_copy(zf_hbm, table.at[rs, :])
        plsc.subcore_barrier()
        tbase = s * PER_TILE

        def main_loop(ids_hbm):
            def load_cps(b, chunk):
                base = tbase + chunk * C
                return (
                    pltpu.make_async_copy(v_hbm.at[pl.ds(base, C), :],
                                          xb[b], sx[b]),
                    pltpu.make_async_copy(ids_hbm.at[pl.ds(base, C)],
                                          ib[b], si[b]),
                )

            def scat_cp(b):
                return pltpu.make_async_copy(xb[b], table.at[ib[b]], ss[b])

            for b in (0, 1):
                for cp in load_cps(b, b):
                    cp.start()

            def round_(g, carry):
                for b in (0, 1):
                    for cp in load_cps(b, 2 * g + b):
                        cp.wait()
                    pltpu.async_copy(xb[b], table.at[ib[b]], ss[b], add=True)
                for b in (0, 1):
                    scat_cp(b).wait()
                    for cp in load_cps(b, 2 * g + 2 + b):
                        cp.start()
                return carry

            lax.fori_loop(0, N_CHUNK // 2 - 1, round_, 0)
            g_last = N_CHUNK // 2 - 1
            for b in (0, 1):
                for cp in load_cps(b, 2 * g_last + b):
                    cp.wait()
                pltpu.async_copy(xb[b], table.at[ib[b]], ss[b], add=True)
            for b in (0, 1):
                scat_cp(b).wait()

        @pl.when(c == 0)
        def _():
            main_loop(rows_hbm)

        @pl.when(c == 1)
        def _():
            main_loop(cols_hbm)

        plsc.subcore_barrier()

        @pl.when(c == 0)
        def _():
            pltpu.sync_copy(table.at[rs, :], rsum_hbm.at[rs, :])

        @pl.when(c == 1)
        def _():
            pltpu.sync_copy(table.at[rs, :], csum_hbm.at[rs, :])

    return pl.kernel(body, out_type=out_type, mesh=mesh, scratch_types=scratch)


def _make_scatter(with_counts: bool):
    mesh = plsc.VectorSubcoreMesh(core_axis_name="c", subcore_axis_name="s")
    out_type = [
        jax.ShapeDtypeStruct((SP, F), jnp.float32),
        jax.ShapeDtypeStruct((SP, F), jnp.float32),
    ]
    scratch = [
        pltpu.VMEM((C, F), jnp.float32),
        pltpu.VMEM((C,), jnp.int32),
        pltpu.VMEM_SHARED((SP, F), jnp.float32),
    ]
    if with_counts:
        out_type.append(jax.ShapeDtypeStruct((SP, 16), jnp.float32))
        out_type.append(jax.ShapeDtypeStruct((SP, 16), jnp.float32))
        scratch.append(pltpu.VMEM((C, 16), jnp.float32))
        scratch.append(pltpu.VMEM_SHARED((SP, 16), jnp.float32))

    def body(v_hbm, rows_hbm, cols_hbm, zf_hbm, zc_hbm, ones_hbm,
             rsum_hbm, csum_hbm, *rest):
        if with_counts:
            cntr_hbm, cntc_hbm, xbuf, ibuf, table, obuf, ctab = rest
        else:
            xbuf, ibuf, table = rest
        c = lax.axis_index("c")
        s = lax.axis_index("s")
        rs = pl.ds(s * SROWS, SROWS)

        pltpu.sync_copy(zf_hbm, table.at[rs, :])
        if with_counts:
            pltpu.sync_copy(zc_hbm, ctab.at[rs, :])
            pltpu.sync_copy(ones_hbm, obuf)
        plsc.subcore_barrier()

        tbase = s * PER_TILE

        def main_loop(ids_hbm):
            def step(it, carry):
                base = tbase + it * C
                pltpu.sync_copy(v_hbm.at[pl.ds(base, C), :], xbuf)
                pltpu.sync_copy(ids_hbm.at[pl.ds(base, C)], ibuf)
                pltpu.sync_copy(xbuf, table.at[ibuf], add=True)
                if with_counts:
                    pltpu.sync_copy(obuf, ctab.at[ibuf], add=True)
                return carry

            lax.fori_loop(0, N_CHUNK, step, 0)

        @pl.when(c == 0)
        def _():
            main_loop(rows_hbm)

        @pl.when(c == 1)
        def _():
            main_loop(cols_hbm)

        plsc.subcore_barrier()

        @pl.when(c == 0)
        def _():
            pltpu.sync_copy(table.at[rs, :], rsum_hbm.at[rs, :])
            if with_counts:
                pltpu.sync_copy(ctab.at[rs, :], cntr_hbm.at[rs, :])

        @pl.when(c == 1)
        def _():
            pltpu.sync_copy(table.at[rs, :], csum_hbm.at[rs, :])
            if with_counts:
                pltpu.sync_copy(ctab.at[rs, :], cntc_hbm.at[rs, :])

    return pl.kernel(body, out_type=out_type, mesh=mesh, scratch_types=scratch)


_ZF = functools.partial(jnp.zeros, (SROWS, F), jnp.float32)
_ZC = functools.partial(jnp.zeros, (SROWS, 16), jnp.float32)
_ONES = functools.partial(jnp.ones, (C, 16), jnp.float32)


def _segment_sums(v, rows, cols, with_counts):
    del with_counts
    fn = _make_scatter_db()
    return fn(v, rows, cols, _ZF())


GW = 32
G_PER = N // GW
G_CHUNK = G_PER // C


def _make_gather(with_residual: bool):
    mesh = plsc.VectorSubcoreMesh(core_axis_name="c", subcore_axis_name="s")
    out_type = jax.ShapeDtypeStruct((N, F), jnp.float32)
    nfeed = 4 if with_residual else 3
    scratch = (
        [pltpu.VMEM((C,), jnp.int32)] * 4
        + [pltpu.VMEM((C, F), jnp.float32)] * 6
        + ([pltpu.VMEM((C, F), jnp.float32)] * 2 if with_residual else [])
        + [pltpu.SemaphoreType.DMA] * (2 * (nfeed + 3))
    )

    def body(xw_hbm, rows_hbm, cols_hbm, ta_hbm, tb_hbm, res_hbm, o_hbm,
             *refs):
        ri = refs[0:2]
        ci = refs[2:4]
        xb = refs[4:6]
        ab = refs[6:8]
        bb = refs[8:10]
        if with_residual:
            rb = refs[10:12]
            sems = refs[12:]
        else:
            sems = refs[10:]
        sri, sci, sxw, sa, sb, so = (sems[0:2], sems[2:4], sems[4:6],
                                     sems[6:8], sems[8:10], sems[10:12])
        if with_residual:
            sr = sems[12:14]
        c = lax.axis_index("c")
        s = lax.axis_index("s")
        w = c * NS + s
        tbase = w * G_PER

        def ids_cps(b, chunk):
            base = tbase + chunk * C
            return (
                pltpu.make_async_copy(rows_hbm.at[pl.ds(base, C)], ri[b], sri[b]),
                pltpu.make_async_copy(cols_hbm.at[pl.ds(base, C)], ci[b], sci[b]),
            )

        def feed_cps(b, chunk):
            base = tbase + chunk * C
            cps = [
                pltpu.make_async_copy(ta_hbm.at[ri[b]], ab[b], sa[b]),
                pltpu.make_async_copy(tb_hbm.at[ci[b]], bb[b], sb[b]),
                pltpu.make_async_copy(xw_hbm.at[pl.ds(base, C), :], xb[b], sxw[b]),
            ]
            if with_residual:
                cps.append(pltpu.make_async_copy(
                    res_hbm.at[pl.ds(base, C), :], rb[b], sr[b]))
            return cps

        def store_cp(b, chunk):
            base = tbase + chunk * C
            return pltpu.make_async_copy(bb[b], o_hbm.at[pl.ds(base, C), :], so[b])

        def compute(b):
            def row(i, carry2):
                for j in range(F // 16):
                    sl = pl.ds(j * 16, 16)
                    v = xb[b][i, sl] + ab[b][i, sl] + bb[b][i, sl]
                    v = jnp.maximum(v, 0.01 * v)
                    if with_residual:
                        v = rb[b][i, sl] + v
                        v = jnp.maximum(v, 0.01 * v)
                    bb[b][i, sl] = v
                return carry2

            lax.fori_loop(0, C, row, 0)

        for b in (0, 1):
            for cp in ids_cps(b, b):
                cp.start()
        for b in (0, 1):
            for cp in ids_cps(b, b):
                cp.wait()
            for cp in feed_cps(b, b):
                cp.start()

        def round_(g, carry):
            for b in (0, 1):
                i = 2 * g + b
                for cp in feed_cps(b, i):
                    cp.wait()
                compute(b)
                store_cp(b, i).start()

                @pl.when(i + 2 < G_CHUNK)
                def _():
                    for cp in ids_cps(b, i + 2):
                        cp.start()

            for b in (0, 1):
                i = 2 * g + b

                @pl.when(i + 2 < G_CHUNK)
                def _():
                    for cp in ids_cps(b, i + 2):
                        cp.wait()
                    store_cp(b, i).wait()
                    for cp in feed_cps(b, i + 2):
                        cp.start()

                @pl.when(i + 2 >= G_CHUNK)
                def _():
                    store_cp(b, i).wait()
            return carry

        lax.fori_loop(0, G_CHUNK // 2, round_, 0)
        if G_CHUNK % 2 == 1:
            i = G_CHUNK - 1
            for cp in feed_cps(0, i):
                cp.wait()
            compute(0)
            store_cp(0, i).start()
            store_cp(0, i).wait()

    return pl.kernel(body, out_type=out_type, mesh=mesh, scratch_types=scratch)


def _gather_epilogue(xw, rows, cols, Ta, Tb, residual):
    fn = _make_gather(residual is not None)
    res = xw if residual is None else residual
    return fn(xw, rows, cols, Ta, Tb, res)


def kernel(input, index, W1, b1, W2, b2):
    x = input
    rows = index[:, 0]
    cols = index[:, 1]

    def tables(rsum, csum, cnt_r, cnt_c, W, b):
        Wr, Wc, Wg = W[:, F:2 * F], W[:, 2 * F:3 * F], W[:, 3 * F:]
        glob = jnp.sum(rsum, axis=0) / N
        Ta = (rsum / cnt_r) @ Wr.T + (glob @ Wg.T + b)[None, :]
        Tb = (csum / cnt_c) @ Wc.T
        return Ta, Tb

    rsum1, csum1 = _segment_sums(x, rows, cols, False)
    rsum1, csum1 = rsum1[:S], csum1[:S]
    ones = jnp.ones((N,), jnp.float32)
    cnt_r = jnp.clip(jax.ops.segment_sum(ones, rows, num_segments=S), 1.0)[:, None]
    cnt_c = jnp.clip(jax.ops.segment_sum(ones, cols, num_segments=S), 1.0)[:, None]
    Ta1, Tb1 = tables(rsum1, csum1, cnt_r, cnt_c, W1, b1)
    xw1 = _mm(x, W1[:, :F])
    step1 = _gather_epilogue(xw1, rows, cols, Ta1, Tb1, None)

    rsum2, csum2 = _segment_sums(step1, rows, cols, False)
    Ta2, Tb2 = tables(rsum2[:S], csum2[:S], cnt_r, cnt_c, W2, b2)
    xw2 = _mm(step1, W2[:, :F])
    return _gather_epilogue(xw2, rows, cols, Ta2, Tb2, x)


# speedup vs baseline: 3.1832x; 1.1203x over previous
import functools

import jax
import jax.numpy as jnp
from jax import lax
from jax.experimental import pallas as pl
from jax.experimental.pallas import tpu as pltpu
from jax.experimental.pallas import tpu_sc as plsc

N = 320000
F = 128
S = 10000

BLK = 2000
C = 80
NS = 16
PER_TILE = N // NS
N_CHUNK = PER_TILE // C
SROWS = 640
SP = NS * SROWS


def _mm_kernel(x_ref, w_ref, o_ref):
    o_ref[...] = jnp.dot(x_ref[...], w_ref[...].T,
                         preferred_element_type=jnp.float32)


def _mm(x, Wx):
    grid = (N // BLK,)
    row_spec = pl.BlockSpec((BLK, F), lambda i: (i, 0))
    w_spec = pl.BlockSpec((F, F), lambda i: (0, 0))
    return pl.pallas_call(
        _mm_kernel,
        grid=grid,
        in_specs=[row_spec, w_spec],
        out_specs=row_spec,
        out_shape=jax.ShapeDtypeStruct((N, F), jnp.float32),
    )(x, Wx)


def _make_scatter_db():
    mesh = plsc.VectorSubcoreMesh(core_axis_name="c", subcore_axis_name="s")
    out_type = [
        jax.ShapeDtypeStruct((SP, F), jnp.float32),
        jax.ShapeDtypeStruct((SP, F), jnp.float32),
    ]
    scratch = (
        [pltpu.VMEM((C, F), jnp.float32)] * 2
        + [pltpu.VMEM((C,), jnp.int32)] * 2
        + [pltpu.VMEM_SHARED((SP, F), jnp.float32)]
        + [pltpu.SemaphoreType.DMA] * 6
    )

    def body(v_hbm, rows_hbm, cols_hbm, zf_hbm, rsum_hbm, csum_hbm,
             xb0, xb1, ib0, ib1, table, sx0, sx1, si0, si1, ss0, ss1):
        c = lax.axis_index("c")
        s = lax.axis_index("s")
        rs = pl.ds(s * SROWS, SROWS)
        xb, ib = (xb0, xb1), (ib0, ib1)
        sx, si, ss = (sx0, sx1), (si0, si1), (ss0, ss1)

        pltpu.sync_copy(zf_hbm, table.at[rs, :])
        plsc.subcore_barrier()
        tbase = s * PER_TILE

        def main_loop(ids_hbm):
            def load_cps(b, chunk):
                base = tbase + chunk * C
                return (
                    pltpu.make_async_copy(v_hbm.at[pl.ds(base, C), :],
                                          xb[b], sx[b]),
                    pltpu.make_async_copy(ids_hbm.at[pl.ds(base, C)],
                                          ib[b], si[b]),
                )

            def scat_cp(b):
                return pltpu.make_async_copy(xb[b], table.at[ib[b]], ss[b])

            for b in (0, 1):
                for cp in load_cps(b, b):
                    cp.start()

            def round_(g, carry):
                for b in (0, 1):
                    for cp in load_cps(b, 2 * g + b):
                        cp.wait()
                    pltpu.async_copy(xb[b], table.at[ib[b]], ss[b], add=True)
                for b in (0, 1):
                    scat_cp(b).wait()
                    for cp in load_cps(b, 2 * g + 2 + b):
                        cp.start()
                return carry

            lax.fori_loop(0, N_CHUNK // 2 - 1, round_, 0)
            g_last = N_CHUNK // 2 - 1
            for b in (0, 1):
                for cp in load_cps(b, 2 * g_last + b):
                    cp.wait()
                pltpu.async_copy(xb[b], table.at[ib[b]], ss[b], add=True)
            for b in (0, 1):
                scat_cp(b).wait()

        @pl.when(c == 0)
        def _():
            main_loop(rows_hbm)

        @pl.when(c == 1)
        def _():
            main_loop(cols_hbm)

        plsc.subcore_barrier()

        @pl.when(c == 0)
        def _():
            pltpu.sync_copy(table.at[rs, :], rsum_hbm.at[rs, :])

        @pl.when(c == 1)
        def _():
            pltpu.sync_copy(table.at[rs, :], csum_hbm.at[rs, :])

    return pl.kernel(body, out_type=out_type, mesh=mesh, scratch_types=scratch)


def _make_scatter(with_counts: bool):
    mesh = plsc.VectorSubcoreMesh(core_axis_name="c", subcore_axis_name="s")
    out_type = [
        jax.ShapeDtypeStruct((SP, F), jnp.float32),
        jax.ShapeDtypeStruct((SP, F), jnp.float32),
    ]
    scratch = [
        pltpu.VMEM((C, F), jnp.float32),
        pltpu.VMEM((C,), jnp.int32),
        pltpu.VMEM_SHARED((SP, F), jnp.float32),
    ]
    if with_counts:
        out_type.append(jax.ShapeDtypeStruct((SP, 16), jnp.float32))
        out_type.append(jax.ShapeDtypeStruct((SP, 16), jnp.float32))
        scratch.append(pltpu.VMEM((C, 16), jnp.float32))
        scratch.append(pltpu.VMEM_SHARED((SP, 16), jnp.float32))

    def body(v_hbm, rows_hbm, cols_hbm, zf_hbm, zc_hbm, ones_hbm,
             rsum_hbm, csum_hbm, *rest):
        if with_counts:
            cntr_hbm, cntc_hbm, xbuf, ibuf, table, obuf, ctab = rest
        else:
            xbuf, ibuf, table = rest
        c = lax.axis_index("c")
        s = lax.axis_index("s")
        rs = pl.ds(s * SROWS, SROWS)

        pltpu.sync_copy(zf_hbm, table.at[rs, :])
        if with_counts:
            pltpu.sync_copy(zc_hbm, ctab.at[rs, :])
            pltpu.sync_copy(ones_hbm, obuf)
        plsc.subcore_barrier()

        tbase = s * PER_TILE

        def main_loop(ids_hbm):
            def step(it, carry):
                base = tbase + it * C
                pltpu.sync_copy(v_hbm.at[pl.ds(base, C), :], xbuf)
                pltpu.sync_copy(ids_hbm.at[pl.ds(base, C)], ibuf)
                pltpu.sync_copy(xbuf, table.at[ibuf], add=True)
                if with_counts:
                    pltpu.sync_copy(obuf, ctab.at[ibuf], add=True)
                return carry

            lax.fori_loop(0, N_CHUNK, step, 0)

        @pl.when(c == 0)
        def _():
            main_loop(rows_hbm)

        @pl.when(c == 1)
        def _():
            main_loop(cols_hbm)

        plsc.subcore_barrier()

        @pl.when(c == 0)
        def _():
            pltpu.sync_copy(table.at[rs, :], rsum_hbm.at[rs, :])
            if with_counts:
                pltpu.sync_copy(ctab.at[rs, :], cntr_hbm.at[rs, :])

        @pl.when(c == 1)
        def _():
            pltpu.sync_copy(table.at[rs, :], csum_hbm.at[rs, :])
            if with_counts:
                pltpu.sync_copy(ctab.at[rs, :], cntc_hbm.at[rs, :])

    return pl.kernel(body, out_type=out_type, mesh=mesh, scratch_types=scratch)


_ZF = functools.partial(jnp.zeros, (SROWS, F), jnp.float32)
_ZC = functools.partial(jnp.zeros, (SROWS, 16), jnp.float32)
_ONES = functools.partial(jnp.ones, (C, 16), jnp.float32)


def _segment_sums(v, rows, cols, with_counts):
    del with_counts
    fn = _make_scatter_db()
    return fn(v, rows, cols, _ZF())


GW = 32
G_PER = N // GW
G_CHUNK = G_PER // C


def _make_gather(with_residual: bool):
    mesh = plsc.VectorSubcoreMesh(core_axis_name="c", subcore_axis_name="s")
    out_type = jax.ShapeDtypeStruct((N, F), jnp.float32)
    nfeed = 4 if with_residual else 3
    scratch = (
        [pltpu.VMEM((C,), jnp.int32)] * 4
        + [pltpu.VMEM((C, F), jnp.float32)] * 8
        + ([pltpu.VMEM((C, F), jnp.float32)] * 2 if with_residual else [])
        + [pltpu.SemaphoreType.DMA] * (2 * (nfeed + 3))
    )

    def body(xw_hbm, rows_hbm, cols_hbm, ta_hbm, tb_hbm, res_hbm, o_hbm,
             *refs):
        ri = refs[0:2]
        ci = refs[2:4]
        xb = refs[4:6]
        ab = refs[6:8]
        bb = refs[8:10]
        ob = refs[10:12]
        if with_residual:
            rb = refs[12:14]
            sems = refs[14:]
        else:
            sems = refs[12:]
        sri, sci, sxw, sa, sb, so = (sems[0:2], sems[2:4], sems[4:6],
                                     sems[6:8], sems[8:10], sems[10:12])
        if with_residual:
            sr = sems[12:14]
        c = lax.axis_index("c")
        s = lax.axis_index("s")
        w = c * NS + s
        tbase = w * G_PER

        def ids_cps(b, chunk):
            base = tbase + chunk * C
            return (
                pltpu.make_async_copy(rows_hbm.at[pl.ds(base, C)], ri[b], sri[b]),
                pltpu.make_async_copy(cols_hbm.at[pl.ds(base, C)], ci[b], sci[b]),
            )

        def feed_cps(b, chunk):
            base = tbase + chunk * C
            cps = [
                pltpu.make_async_copy(ta_hbm.at[ri[b]], ab[b], sa[b]),
                pltpu.make_async_copy(tb_hbm.at[ci[b]], bb[b], sb[b]),
                pltpu.make_async_copy(xw_hbm.at[pl.ds(base, C), :], xb[b], sxw[b]),
            ]
            if with_residual:
                cps.append(pltpu.make_async_copy(
                    res_hbm.at[pl.ds(base, C), :], rb[b], sr[b]))
            return cps

        def store_cp(b, chunk):
            base = tbase + chunk * C
            return pltpu.make_async_copy(ob[b], o_hbm.at[pl.ds(base, C), :], so[b])

        def compute(b):
            def row(i, carry2):
                for j in range(F // 16):
                    sl = pl.ds(j * 16, 16)
                    v = xb[b][i, sl] + ab[b][i, sl] + bb[b][i, sl]
                    v = jnp.maximum(v, 0.01 * v)
                    if with_residual:
                        v = rb[b][i, sl] + v
                        v = jnp.maximum(v, 0.01 * v)
                    ob[b][i, sl] = v
                return carry2

            lax.fori_loop(0, C, row, 0)

        for b in (0, 1):
            for cp in ids_cps(b, b):
                cp.start()
        for b in (0, 1):
            for cp in ids_cps(b, b):
                cp.wait()
            for cp in feed_cps(b, b):
                cp.start()

        def round_(g, carry):
            for b in (0, 1):
                i = 2 * g + b
                for cp in feed_cps(b, i):
                    cp.wait()

                @pl.when(i + 2 < G_CHUNK)
                def _():
                    for cp in ids_cps(b, i + 2):
                        cp.start()

                @pl.when(i >= 2)
                def _():
                    store_cp(b, i - 2).wait()

                compute(b)
                store_cp(b, i).start()

                @pl.when(i + 2 < G_CHUNK)
                def _():
                    for cp in ids_cps(b, i + 2):
                        cp.wait()
                    for cp in feed_cps(b, i + 2):
                        cp.start()
            return carry

        lax.fori_loop(0, G_CHUNK // 2, round_, 0)
        last_even = 2 * (G_CHUNK // 2)
        if G_CHUNK % 2 == 1:
            i = G_CHUNK - 1
            for cp in feed_cps(0, i):
                cp.wait()
            store_cp(0, i - 2).wait()
            compute(0)
            store_cp(0, i).start()
            store_cp(0, i).wait()
            store_cp(1, last_even - 1).wait()
        else:
            store_cp(0, last_even - 2).wait()
            store_cp(1, last_even - 1).wait()

    return pl.kernel(body, out_type=out_type, mesh=mesh, scratch_types=scratch)


def _gather_epilogue(xw, rows, cols, Ta, Tb, residual):
    fn = _make_gather(residual is not None)
    res = xw if residual is None else residual
    return fn(xw, rows, cols, Ta, Tb, res)


def kernel(input, index, W1, b1, W2, b2):
    x = input
    rows = index[:, 0]
    cols = index[:, 1]

    def tables(rsum, csum, cnt_r, cnt_c, W, b):
        Wr, Wc, Wg = W[:, F:2 * F], W[:, 2 * F:3 * F], W[:, 3 * F:]
        glob = jnp.sum(rsum, axis=0) / N
        Ta = (rsum / cnt_r) @ Wr.T + (glob @ Wg.T + b)[None, :]
        Tb = (csum / cnt_c) @ Wc.T
        return Ta, Tb

    rsum1, csum1 = _segment_sums(x, rows, cols, False)
    rsum1, csum1 = rsum1[:S], csum1[:S]
    ones = jnp.ones((N,), jnp.float32)
    cnt_r = jnp.clip(jax.ops.segment_sum(ones, rows, num_segments=S), 1.0)[:, None]
    cnt_c = jnp.clip(jax.ops.segment_sum(ones, cols, num_segments=S), 1.0)[:, None]
    Ta1, Tb1 = tables(rsum1, csum1, cnt_r, cnt_c, W1, b1)
    xw1 = _mm(x, W1[:, :F])
    step1 = _gather_epilogue(xw1, rows, cols, Ta1, Tb1, None)

    rsum2, csum2 = _segment_sums(step1, rows, cols, False)
    Ta2, Tb2 = tables(rsum2[:S], csum2[:S], cnt_r, cnt_c, W2, b2)
    xw2 = _mm(step1, W2[:, :F])
    return _gather_epilogue(xw2, rows, cols, Ta2, Tb2, x)
